# baseline (speedup 1.0000x reference)
import functools
import jax
import jax.numpy as jnp
from jax import lax
from jax.experimental import pallas as pl
from jax.experimental.pallas import tpu as pltpu
from jax.experimental.pallas import tpu_sc as plsc

_NSEQ = 2
_NBUF = 2


def _pe_add_kernel(B, S, D):
    info = plsc.get_sparse_core_info()
    NC, NS, L = info.num_cores, info.num_subcores, info.num_lanes
    NW = NC * NS
    assert B % NW == 0 and D % L == 0
    seqs_per_w = B // NW
    assert seqs_per_w % _NSEQ == 0
    n_chunks = seqs_per_w // _NSEQ
    assert n_chunks >= 2 * _NBUF and n_chunks % _NBUF == 0

    mesh = plsc.VectorSubcoreMesh(core_axis_name="c", subcore_axis_name="s")
    buf_t = pltpu.VMEM((_NSEQ, S, D), jnp.float32)

    @functools.partial(
        pl.kernel,
        out_type=jax.ShapeDtypeStruct((B, S, D), jnp.float32),
        mesh=mesh,
        compiler_params=pltpu.CompilerParams(use_tc_tiling_on_sc=False),
        scratch_types=[
            pltpu.VMEM((S, D), jnp.float32),
            [buf_t] * _NBUF,
            [buf_t] * _NBUF,
            [pltpu.SemaphoreType.DMA] * _NBUF,
            [pltpu.SemaphoreType.DMA] * _NBUF,
        ],
    )
    def _k(x_hbm, pe_hbm, out_hbm, pe_v, ibufs, obufs, isems, osems):
        wid = lax.axis_index("s") * NC + lax.axis_index("c")
        pltpu.sync_copy(pe_hbm.at[pl.ds(0, S)], pe_v)
        base = wid * seqs_per_w

        def start_in(c, b):
            pltpu.async_copy(
                x_hbm.at[pl.ds(base + c * _NSEQ, _NSEQ)], ibufs[b], isems[b])

        def wait_in(c, b):
            pltpu.make_async_copy(
                x_hbm.at[pl.ds(base + c * _NSEQ, _NSEQ)], ibufs[b],
                isems[b]).wait()

        def start_out(c, b):
            pltpu.async_copy(
                obufs[b], out_hbm.at[pl.ds(base + c * _NSEQ, _NSEQ)], osems[b])

        def wait_out(c, b):
            pltpu.make_async_copy(
                obufs[b], out_hbm.at[pl.ds(base + c * _NSEQ, _NSEQ)],
                osems[b]).wait()

        def compute(b):
            @pl.loop(0, S, unroll=4)
            def _row(s):
                for rep in range(_NSEQ):
                    for q in range(D // L):
                        sl = pl.ds(q * L, L)
                        obufs[b][rep, s, sl] = ibufs[b][rep, s, sl] + pe_v[s, sl]

        for b in range(_NBUF):
            start_in(b, b)

        for c in range(_NBUF):
            b = c % _NBUF
            wait_in(c, b)
            compute(b)
            start_out(c, b)
            start_in(c + _NBUF, b)

        @pl.loop(_NBUF, n_chunks - _NBUF, step=_NBUF)
        def _main(ci):
            for b in range(_NBUF):
                c = ci + b
                wait_in(c, b)
                wait_out(c - _NBUF, b)
                compute(b)
                start_out(c, b)
                start_in(c + _NBUF, b)

        for cc in range(n_chunks - _NBUF, n_chunks):
            b = cc % _NBUF
            wait_in(cc, b)
            wait_out(cc - _NBUF, b)
            compute(b)
            start_out(cc, b)

        for cc in range(n_chunks - _NBUF, n_chunks):
            wait_out(cc, cc % _NBUF)

    return _k


def kernel(x, pe_weight):
    B, S, D = x.shape
    return _pe_add_kernel(B, S, D)(x, pe_weight)


# speedup vs baseline: 1.2736x; 1.2736x over previous
import functools
import jax
import jax.numpy as jnp
from jax import lax
from jax.experimental import pallas as pl
from jax.experimental.pallas import tpu as pltpu
from jax.experimental.pallas import tpu_sc as plsc

_NSEQ = 2
_RING = 4


def _pe_add_kernel(B, R, W, L):
    info = plsc.get_sparse_core_info()
    NC, NS = info.num_cores, info.num_subcores
    NW = NC * NS
    assert B % NW == 0 and W % L == 0
    seqs_per_w = B // NW
    assert seqs_per_w % _NSEQ == 0
    n_chunks = seqs_per_w // _NSEQ
    assert n_chunks >= 2 * _RING and n_chunks % _RING == 0

    mesh = plsc.VectorSubcoreMesh(core_axis_name="c", subcore_axis_name="s")

    @functools.partial(
        pl.kernel,
        out_type=jax.ShapeDtypeStruct((B, R, W), jnp.float32),
        mesh=mesh,
        compiler_params=pltpu.CompilerParams(use_tc_tiling_on_sc=False),
        scratch_types=[
            pltpu.VMEM((R, W), jnp.float32),
            [pltpu.VMEM((_NSEQ, R, W), jnp.float32)] * _RING,
            [pltpu.SemaphoreType.DMA] * _RING,
            [pltpu.SemaphoreType.DMA] * _RING,
        ],
    )
    def _k(x_hbm, pe_hbm, out_hbm, pe_v, bufs, isems, osems):
        wid = lax.axis_index("s") * NC + lax.axis_index("c")
        pltpu.sync_copy(pe_hbm, pe_v)
        base = wid * seqs_per_w

        def start_in(c, b):
            pltpu.async_copy(
                x_hbm.at[pl.ds(base + c * _NSEQ, _NSEQ)], bufs[b], isems[b])

        def wait_in(c, b):
            pltpu.make_async_copy(
                x_hbm.at[pl.ds(base + c * _NSEQ, _NSEQ)], bufs[b],
                isems[b]).wait()

        def start_out(c, b):
            pltpu.async_copy(
                bufs[b], out_hbm.at[pl.ds(base + c * _NSEQ, _NSEQ)], osems[b])

        def wait_out(c, b):
            pltpu.make_async_copy(
                bufs[b], out_hbm.at[pl.ds(base + c * _NSEQ, _NSEQ)],
                osems[b]).wait()

        def compute(b):
            @pl.loop(0, R, unroll=4)
            def _row(s):
                for rep in range(_NSEQ):
                    for q in range(W // L):
                        sl = pl.ds(q * L, L)
                        bufs[b][rep, s, sl] = bufs[b][rep, s, sl] + pe_v[s, sl]

        PF = _RING - 2

        for c in range(PF):
            start_in(c, c % _RING)

        for c in range(PF):
            b = c % _RING
            wait_in(c, b)
            compute(b)
            start_out(c, b)
            start_in(c + PF, (c + PF) % _RING)

        @pl.loop(PF, n_chunks - PF, step=_RING)
        def _main(ci):
            for j in range(_RING):
                c = ci + j
                b = (PF + j) % _RING
                wait_in(c, b)
                compute(b)
                start_out(c, b)
                wait_out(c - PF, j % _RING)
                start_in(c + PF, j % _RING)

        for cc in range(n_chunks - PF, n_chunks):
            b = cc % _RING
            wait_in(cc, b)
            compute(b)
            start_out(cc, b)
            wait_out(cc - PF, (cc - PF) % _RING)

        for cc in range(n_chunks - PF, n_chunks):
            wait_out(cc, cc % _RING)

    return _k


def kernel(x, pe_weight):
    B, S, D = x.shape
    W = 128
    assert (S * D) % W == 0
    R = (S * D) // W
    x2 = x.reshape(B, R, W)
    pe2 = pe_weight[:S].reshape(R, W)
    out = _pe_add_kernel(B, R, W, 16)(x2, pe2)
    return out.reshape(B, S, D)


# speedup vs baseline: 1.4204x; 1.1153x over previous
import functools
import jax
import jax.numpy as jnp
from jax import lax
from jax.experimental import pallas as pl
from jax.experimental.pallas import tpu as pltpu
from jax.experimental.pallas import tpu_sc as plsc

_L = 16


def _pe_add_kernel(B, R, W, nseq, ring, tc_tiling):
    info = plsc.get_sparse_core_info()
    NC, NS = info.num_cores, info.num_subcores
    NW = NC * NS
    assert B % NW == 0 and W % _L == 0
    seqs_per_w = B // NW
    assert seqs_per_w % nseq == 0
    n_chunks = seqs_per_w // nseq
    PF = ring - 2
    assert PF >= 1 and (n_chunks - 2 * PF) % ring == 0

    mesh = plsc.VectorSubcoreMesh(core_axis_name="c", subcore_axis_name="s")

    @functools.partial(
        pl.kernel,
        out_type=jax.ShapeDtypeStruct((B, R, W), jnp.float32),
        mesh=mesh,
        compiler_params=pltpu.CompilerParams(use_tc_tiling_on_sc=tc_tiling),
        scratch_types=[
            pltpu.VMEM((R, W), jnp.float32),
            [pltpu.VMEM((nseq, R, W), jnp.float32)] * ring,
            [pltpu.SemaphoreType.DMA] * ring,
            [pltpu.SemaphoreType.DMA] * ring,
        ],
    )
    def _k(x_hbm, pe_hbm, out_hbm, pe_v, bufs, isems, osems):
        wid = lax.axis_index("s") * NC + lax.axis_index("c")
        pltpu.sync_copy(pe_hbm, pe_v)
        base = wid * seqs_per_w

        def start_in(c, b):
            pltpu.async_copy(
                x_hbm.at[pl.ds(base + c * nseq, nseq)], bufs[b], isems[b])

        def wait_in(c, b):
            pltpu.make_async_copy(
                x_hbm.at[pl.ds(base + c * nseq, nseq)], bufs[b],
                isems[b]).wait()

        def start_out(c, b):
            pltpu.async_copy(
                bufs[b], out_hbm.at[pl.ds(base + c * nseq, nseq)], osems[b])

        def wait_out(c, b):
            pltpu.make_async_copy(
                bufs[b], out_hbm.at[pl.ds(base + c * nseq, nseq)],
                osems[b]).wait()

        def compute(b):
            @pl.loop(0, R, unroll=4)
            def _row(s):
                for rep in range(nseq):
                    for q in range(W // _L):
                        sl = pl.ds(q * _L, _L)
                        bufs[b][rep, s, sl] = bufs[b][rep, s, sl] + pe_v[s, sl]

        for c in range(PF):
            start_in(c, c % ring)

        for c in range(PF):
            b = c % ring
            wait_in(c, b)
            compute(b)
            start_out(c, b)
            start_in(c + PF, (c + PF) % ring)

        @pl.loop(PF, n_chunks - PF, step=ring)
        def _main(ci):
            for j in range(ring):
                c = ci + j
                wait_in(c, (PF + j) % ring)
                compute((PF + j) % ring)
                start_out(c, (PF + j) % ring)
                wait_out(c - PF, j % ring)
                start_in(c + PF, j % ring)

        for cc in range(n_chunks - PF, n_chunks):
            b = cc % ring
            wait_in(cc, b)
            compute(b)
            start_out(cc, b)
            wait_out(cc - PF, (cc - PF) % ring)

        for cc in range(n_chunks - PF, n_chunks):
            wait_out(cc, cc % ring)

    return _k


def kernel(x, pe_weight):
    B, S, D = x.shape
    pe = pe_weight[:S]
    out = _pe_add_kernel(B, S, D, 1, 4, True)(x, pe)
    return out
